# baseline (speedup 1.0000x reference)
import functools

import jax
import jax.numpy as jnp
from jax import lax
from jax.experimental import pallas as pl
from jax.experimental.pallas import tpu as pltpu
from jax.experimental.pallas import tpu_sc as plsc

NC = 2
NS = 16
LANES = 16
NW = NC * NS
EPS = 1e-12


def _rsqrt16(v):
    i = lax.bitcast_convert_type(v, jnp.int32)
    i = jnp.int32(0x5F3759DF) - lax.shift_right_logical(i, 1)
    y = lax.bitcast_convert_type(i, jnp.float32)
    for _ in range(2):
        y = y * (1.5 - 0.5 * v * y * y)
    return y


@functools.cache
def _build(n_chunks, C, SEQ, H, per_w):
    mesh = plsc.VectorSubcoreMesh(core_axis_name="c", subcore_axis_name="s")
    grp = C // LANES
    h8 = H // LANES

    @functools.partial(
        pl.kernel,
        out_type=(
            jax.ShapeDtypeStruct((NW * per_w, H), jnp.float32),
            jax.ShapeDtypeStruct((NW * per_w,), jnp.int32),
        ),
        mesh=mesh,
        compiler_params=pltpu.CompilerParams(needs_layout_passes=False),
        scratch_types=[
            pltpu.VMEM((n_chunks, C), jnp.int32),
            pltpu.VMEM((SEQ + C, H), jnp.float32),
            pltpu.VMEM((C, H), jnp.float32),
            pltpu.VMEM((C, H), jnp.float32),
            pltpu.VMEM((per_w,), jnp.int32),
            pltpu.VMEM((H,), jnp.float32),
            pltpu.VMEM((H,), jnp.float32),
            pltpu.VMEM((H,), jnp.float32),
            pltpu.SemaphoreType.DMA,
            pltpu.SemaphoreType.DMA,
            pltpu.SemaphoreType.DMA,
            pltpu.SemaphoreType.DMA,
            pltpu.SemaphoreType.DMA,
        ],
    )
    def k(ids_hbm, word_hbm, pos_hbm, tok_hbm, gamma_hbm, beta_hbm,
          emb_out, pos_out,
          idx_all, comb, buf0, buf1, posbuf, tokrow,
          gamma_v, beta_v, g0, g1, o0, o1, psem):
        wid = lax.axis_index("s") * NC + lax.axis_index("c")
        base_tok = wid * per_w

        pltpu.sync_copy(ids_hbm.at[wid], idx_all)
        pltpu.sync_copy(pos_hbm.at[pl.ds(0, SEQ)], comb.at[pl.ds(0, SEQ)])
        pltpu.sync_copy(tok_hbm.at[0], tokrow)
        pltpu.sync_copy(gamma_hbm, gamma_v)
        pltpu.sync_copy(beta_hbm, beta_v)

        iota16 = lax.iota(jnp.int32, 16)

        @pl.loop(0, SEQ)
        def _(r):
            for c8 in range(h8):
                sl = pl.ds(c8 * LANES, LANES)
                comb[r, sl] = comb[r, sl] + tokrow[sl]

        @pl.loop(0, C)
        def _(r):
            for c8 in range(h8):
                sl = pl.ds(c8 * LANES, LANES)
                comb[SEQ + r, sl] = comb[r, sl]

        @pl.loop(0, per_w // LANES)
        def _(g):
            posbuf[pl.ds(g * LANES, LANES)] = (g * LANES + iota16) % SEQ

        pltpu.async_copy(posbuf, pos_out.at[pl.ds(base_tok, per_w)], psem)

        bufs = (buf0, buf1)
        gsems = (g0, g1)
        osems = (o0, o1)

        def start_gather(j, b):
            pltpu.async_copy(word_hbm.at[idx_all.at[j]], bufs[b], gsems[b])

        def wait_gather(j, b):
            pltpu.make_async_copy(
                word_hbm.at[idx_all.at[j]], bufs[b], gsems[b]).wait()

        def out_ref(j):
            return emb_out.at[pl.ds(base_tok + j * C, C)]

        def start_out(j, b):
            pltpu.async_copy(bufs[b], out_ref(j), osems[b])

        def wait_out(j, b):
            pltpu.make_async_copy(bufs[b], out_ref(j), osems[b]).wait()

        start_gather(0, 0)

        gvs = tuple(gamma_v[pl.ds(kk * LANES, LANES)] for kk in range(h8))
        bvs = tuple(beta_v[pl.ds(kk * LANES, LANES)] for kk in range(h8))

        gb_id = jnp.bool_(True)
        for kk in range(h8):
            gb_id = gb_id & jnp.all(gvs[kk] == 1.0) & jnp.all(bvs[kk] == 0.0)

        perms = tuple(iota16 ^ (1 << kk) for kk in range(4))

        def lane_sum(v):
            for pvec in perms:
                v = v + jnp.take(v, pvec)
            return v

        def row_norm(bufb, pos_base, r, affine):
            p = pos_base + r
            xs = [bufb[r, pl.ds(c8 * LANES, LANES)]
                  + comb[p, pl.ds(c8 * LANES, LANES)]
                  for c8 in range(h8)]
            sv = ((xs[0] + xs[1]) + (xs[2] + xs[3])) \
                + ((xs[4] + xs[5]) + (xs[6] + xs[7]))
            qv = ((xs[0] * xs[0] + xs[1] * xs[1])
                  + (xs[2] * xs[2] + xs[3] * xs[3])) \
                + ((xs[4] * xs[4] + xs[5] * xs[5])
                   + (xs[6] * xs[6] + xs[7] * xs[7]))
            mean = lane_sum(sv) * (1.0 / H)
            var = lane_sum(qv) * (1.0 / H) - mean * mean
            scale = _rsqrt16(var + EPS)
            nm = mean * scale
            for c8 in range(h8):
                y = xs[c8] * scale - nm
                if affine:
                    y = y * gvs[c8] + bvs[c8]
                bufb[r, pl.ds(c8 * LANES, LANES)] = y

        def process(j, b):
            nj = j + 1
            nb = 1 - b

            @pl.when(nj < n_chunks)
            def _():
                @pl.when(nj >= 2)
                def _():
                    wait_out(nj - 2, nb)
                start_gather(nj, nb)

            wait_gather(j, b)
            bufb = bufs[b]
            pos_base = (j * C) % SEQ

            @pl.when(gb_id)
            def _():
                @plsc.parallel_loop(0, C)
                def _(r):
                    row_norm(bufb, pos_base, r, affine=False)

            @pl.when(jnp.logical_not(gb_id))
            def _():
                @plsc.parallel_loop(0, C, unroll=4)
                def _(r):
                    row_norm(bufb, pos_base, r, affine=True)

            start_out(j, b)

        @pl.loop(0, n_chunks, step=2)
        def _(j):
            process(j, 0)
            process(j + 1, 1)

        wait_out(n_chunks - 2, 0)
        wait_out(n_chunks - 1, 1)
        pltpu.make_async_copy(
            posbuf, pos_out.at[pl.ds(base_tok, per_w)], psem).wait()

    return k


def kernel(input_ids, word_emb, pos_emb, tok_emb, gamma, beta):
    S0, B, L = input_ids.shape
    H = word_emb.shape[1]
    N = S0 * B * L
    per_w = N // NW
    C = 128
    n_chunks = per_w // C

    ids3 = input_ids.reshape(NW, n_chunks, C).astype(jnp.int32)
    k = _build(n_chunks, C, L, H, per_w)
    emb_flat, pos_flat = k(ids3, word_emb, pos_emb, tok_emb, gamma, beta)
    emb = emb_flat.reshape(S0, B, L, H)
    pos = pos_flat.reshape(S0, B, L).astype(input_ids.dtype)
    return (emb, pos)


# speedup vs baseline: 1.3774x; 1.3774x over previous
import functools

import jax
import jax.numpy as jnp
from jax import lax
from jax.experimental import pallas as pl
from jax.experimental.pallas import tpu as pltpu
from jax.experimental.pallas import tpu_sc as plsc

NC = 2
NS = 16
LANES = 16
NW = NC * NS
EPS = 1e-12
NBUF = 4


def _rsqrt16(v):
    i = lax.bitcast_convert_type(v, jnp.int32)
    i = jnp.int32(0x5F3759DF) - lax.shift_right_logical(i, 1)
    y = lax.bitcast_convert_type(i, jnp.float32)
    for _ in range(2):
        y = y * (1.5 - 0.5 * v * y * y)
    return y


@functools.cache
def _build(n_chunks, C, SEQ, H, per_w):
    mesh = plsc.VectorSubcoreMesh(core_axis_name="c", subcore_axis_name="s")
    h8 = H // LANES

    @functools.partial(
        pl.kernel,
        out_type=(
            jax.ShapeDtypeStruct((NW * per_w, H), jnp.float32),
            jax.ShapeDtypeStruct((NW * per_w,), jnp.int32),
        ),
        mesh=mesh,
        compiler_params=pltpu.CompilerParams(needs_layout_passes=False),
        scratch_types=[
            pltpu.VMEM((n_chunks, C), jnp.int32),
            pltpu.VMEM((SEQ, H), jnp.float32),
            pltpu.VMEM((NBUF, C, H), jnp.float32),
            pltpu.VMEM((per_w,), jnp.int32),
            pltpu.VMEM((H,), jnp.float32),
            pltpu.VMEM((H,), jnp.float32),
            pltpu.VMEM((H,), jnp.float32),
            pltpu.VMEM_SHARED((SEQ + C, H), jnp.float32),
            pltpu.SemaphoreType.DMA,
            pltpu.SemaphoreType.DMA,
            pltpu.SemaphoreType.DMA,
            pltpu.SemaphoreType.DMA,
            pltpu.SemaphoreType.DMA,
            pltpu.SemaphoreType.DMA,
            pltpu.SemaphoreType.DMA,
            pltpu.SemaphoreType.DMA,
            pltpu.SemaphoreType.DMA,
            pltpu.SemaphoreType.DMA,
            pltpu.SemaphoreType.DMA,
            pltpu.SemaphoreType.DMA,
            pltpu.SemaphoreType.DMA,
        ],
    )
    def k(ids_hbm, word_hbm, pos_hbm, tok_hbm, gamma_hbm, beta_hbm,
          emb_out, pos_out,
          idx_all, cstage, bufs, posbuf, tokrow, gamma_v, beta_v, comb_sh,
          pf0, pf1, pf2, pf3, g0, g1, g2, g3, o0, o1, o2, o3, psem):
        cid = lax.axis_index("c")
        sid = lax.axis_index("s")
        wid = sid * NC + cid
        base_tok = wid * per_w

        pltpu.sync_copy(ids_hbm.at[wid], idx_all)
        pltpu.sync_copy(gamma_hbm, gamma_v)
        pltpu.sync_copy(beta_hbm, beta_v)

        iota16 = lax.iota(jnp.int32, 16)

        @pl.when(sid == 0)
        def _():
            pltpu.sync_copy(pos_hbm.at[pl.ds(0, SEQ)], cstage)
            pltpu.sync_copy(tok_hbm.at[0], tokrow)

            @pl.loop(0, SEQ)
            def _(r):
                for c8 in range(h8):
                    sl = pl.ds(c8 * LANES, LANES)
                    cstage[r, sl] = cstage[r, sl] + tokrow[sl]

            pltpu.sync_copy(cstage, comb_sh.at[pl.ds(0, SEQ)])
            pltpu.sync_copy(cstage.at[pl.ds(0, C)],
                            comb_sh.at[pl.ds(SEQ, C)])

        @pl.loop(0, per_w // LANES)
        def _(g):
            posbuf[pl.ds(g * LANES, LANES)] = (g * LANES + iota16) % SEQ

        pltpu.async_copy(posbuf, pos_out.at[pl.ds(base_tok, per_w)], psem)

        plsc.subcore_barrier()

        pfsems = (pf0, pf1, pf2, pf3)
        gsems = (g0, g1, g2, g3)
        osems = (o0, o1, o2, o3)

        def pos_base(j):
            return (j * C) % SEQ

        def start_prefill(j, b):
            pltpu.async_copy(comb_sh.at[pl.ds(pos_base(j), C)],
                             bufs.at[b], pfsems[b])

        def wait_prefill(j, b):
            pltpu.make_async_copy(comb_sh.at[pl.ds(pos_base(j), C)],
                                  bufs.at[b], pfsems[b]).wait()

        def start_gather(j, b):
            pltpu.async_copy(word_hbm.at[idx_all.at[j]], bufs.at[b],
                             gsems[b], add=True)

        def wait_gather(j, b):
            pltpu.make_async_copy(
                word_hbm.at[idx_all.at[j]], bufs.at[b], gsems[b]).wait()

        def out_ref(j):
            return emb_out.at[pl.ds(base_tok + j * C, C)]

        def start_out(j, b):
            pltpu.async_copy(bufs.at[b], out_ref(j), osems[b])

        def wait_out(j, b):
            pltpu.make_async_copy(bufs.at[b], out_ref(j), osems[b]).wait()

        start_prefill(0, 0)
        start_prefill(1, 1)
        wait_prefill(0, 0)
        start_gather(0, 0)

        gvs = tuple(gamma_v[pl.ds(kk * LANES, LANES)] for kk in range(h8))
        bvs = tuple(beta_v[pl.ds(kk * LANES, LANES)] for kk in range(h8))

        gb_id = jnp.bool_(True)
        for kk in range(h8):
            gb_id = gb_id & jnp.all(gvs[kk] == 1.0) & jnp.all(bvs[kk] == 0.0)

        perms = tuple(iota16 ^ (1 << kk) for kk in range(4))

        def lane_sum(v):
            for pvec in perms:
                v = v + jnp.take(v, pvec)
            return v

        def row_norm(bufb, r, affine):
            xs = [bufb[r, pl.ds(c8 * LANES, LANES)] for c8 in range(h8)]
            sv = ((xs[0] + xs[1]) + (xs[2] + xs[3])) \
                + ((xs[4] + xs[5]) + (xs[6] + xs[7]))
            qv = ((xs[0] * xs[0] + xs[1] * xs[1])
                  + (xs[2] * xs[2] + xs[3] * xs[3])) \
                + ((xs[4] * xs[4] + xs[5] * xs[5])
                   + (xs[6] * xs[6] + xs[7] * xs[7]))
            mean = lane_sum(sv) * (1.0 / H)
            var = lane_sum(qv) * (1.0 / H) - mean * mean
            scale = _rsqrt16(var + EPS)
            nm = mean * scale
            for c8 in range(h8):
                y = xs[c8] * scale - nm
                if affine:
                    y = y * gvs[c8] + bvs[c8]
                bufb[r, pl.ds(c8 * LANES, LANES)] = y

        def process(j, b):
            jp = j + 2
            bp = (b + 2) % NBUF

            @pl.when(jp < n_chunks)
            def _():
                @pl.when(j >= 2)
                def _():
                    wait_out(j - 2, bp)
                start_prefill(jp, bp)

            jg = j + 1
            bg = (b + 1) % NBUF

            @pl.when(jg < n_chunks)
            def _():
                wait_prefill(jg, bg)
                start_gather(jg, bg)

            wait_gather(j, b)
            bufb = bufs.at[b]

            @pl.when(gb_id)
            def _():
                @plsc.parallel_loop(0, C, unroll=2)
                def _(r):
                    row_norm(bufb, r, affine=False)

            @pl.when(jnp.logical_not(gb_id))
            def _():
                @plsc.parallel_loop(0, C, unroll=2)
                def _(r):
                    row_norm(bufb, r, affine=True)

            start_out(j, b)

        @pl.loop(0, n_chunks, step=NBUF)
        def _(j):
            for b in range(NBUF):
                process(j + b, b)

        for b in range(NBUF):
            wait_out(n_chunks - NBUF + b, b)
        pltpu.make_async_copy(
            posbuf, pos_out.at[pl.ds(base_tok, per_w)], psem).wait()

    return k


def kernel(input_ids, word_emb, pos_emb, tok_emb, gamma, beta):
    S0, B, L = input_ids.shape
    H = word_emb.shape[1]
    N = S0 * B * L
    per_w = N // NW
    C = 128
    n_chunks = per_w // C

    ids3 = input_ids.reshape(NW, n_chunks, C).astype(jnp.int32)
    k = _build(n_chunks, C, L, H, per_w)
    emb_flat, pos_flat = k(ids3, word_emb, pos_emb, tok_emb, gamma, beta)
    emb = emb_flat.reshape(S0, B, L, H)
    pos = pos_flat.reshape(S0, B, L).astype(input_ids.dtype)
    return (emb, pos)
